# baseline (speedup 1.0000x reference)
import functools

import jax
import jax.numpy as jnp
from jax import lax
from jax.experimental import pallas as pl
from jax.experimental.pallas import tpu as pltpu
from jax.experimental.pallas import tpu_sc as plsc

B = 16384
D = 32
IDX_CHUNK = 128


@functools.lru_cache(maxsize=None)
def _make_gather(num_cores, num_subcores):
    NC, NS = num_cores, num_subcores
    NW = NC * NS
    b_per_w = B // NW
    n_chunks = b_per_w // IDX_CHUNK
    mesh = plsc.VectorSubcoreMesh(core_axis_name="c", subcore_axis_name="s")

    @functools.partial(
        pl.kernel,
        mesh=mesh,
        compiler_params=pltpu.CompilerParams(use_tc_tiling_on_sc=False),
        out_type=[
            jax.ShapeDtypeStruct((B, D), jnp.float32),
            jax.ShapeDtypeStruct((B, D), jnp.float32),
        ],
        scratch_types=[
            pltpu.VMEM((n_chunks, IDX_CHUNK), jnp.int32),
            pltpu.VMEM((b_per_w, D), jnp.float32),
            pltpu.VMEM((n_chunks, IDX_CHUNK), jnp.int32),
            pltpu.VMEM((b_per_w, D), jnp.float32),
            pltpu.SemaphoreType.DMA,
            pltpu.SemaphoreType.DMA,
        ],
    )
    def gather_k(user_tab, movie_tab, uid, mid, out_u, out_m,
                 uidx, urows, midx, mrows, usem, msem):
        wid = lax.axis_index("s") * NC + lax.axis_index("c")
        base = wid * b_per_w
        pltpu.sync_copy(uid.at[wid], uidx)
        pltpu.sync_copy(mid.at[wid], midx)
        copies = []
        for j in range(n_chunks):
            copies.append(pltpu.async_copy(
                user_tab.at[uidx.at[j]],
                urows.at[pl.ds(j * IDX_CHUNK, IDX_CHUNK)], usem))
            copies.append(pltpu.async_copy(
                movie_tab.at[midx.at[j]],
                mrows.at[pl.ds(j * IDX_CHUNK, IDX_CHUNK)], msem))
        for c in copies:
            c.wait()
        pltpu.sync_copy(urows, out_u.at[pl.ds(base, b_per_w)])
        pltpu.sync_copy(mrows, out_m.at[pl.ds(base, b_per_w)])

    return gather_k


BLK = 2048


def _mlp_body(u_ref, m_ref, c_ref, g_ref, a_ref, o_ref,
              gt_ref, at_ref, ot_ref,
              w0_ref, b0_ref, w1_ref, b1_ref, w2_ref, b2_ref,
              w3_ref, b3_ref, out_ref):
    f32 = jnp.float32
    acc = jnp.dot(u_ref[...], w0_ref[0:32, :], preferred_element_type=f32)
    acc += jnp.dot(m_ref[...], w0_ref[32:64, :], preferred_element_type=f32)
    acc += jnp.dot(c_ref[...], w0_ref[88:104, :], preferred_element_type=f32)

    def small(idx_ref, tab_ref, lo, hi, T):
        oh = (idx_ref[...] ==
              lax.broadcasted_iota(jnp.int32, (BLK, T), 1)).astype(f32)
        e = jnp.dot(oh, tab_ref[...], preferred_element_type=f32)
        return jnp.dot(e, w0_ref[lo:hi, :], preferred_element_type=f32)

    acc += small(g_ref, gt_ref, 64, 72, 2)
    acc += small(a_ref, at_ref, 72, 80, 7)
    acc += small(o_ref, ot_ref, 80, 88, 21)
    h = jnp.maximum(acc + b0_ref[...], 0.0)
    h = jnp.maximum(jnp.dot(h, w1_ref[...], preferred_element_type=f32)
                    + b1_ref[...], 0.0)
    h = jnp.maximum(jnp.dot(h, w2_ref[...], preferred_element_type=f32)
                    + b2_ref[...], 0.0)
    out_ref[...] = (jnp.dot(h, w3_ref[...], preferred_element_type=f32)
                    + b3_ref[...])


def _full(shape):
    return pl.BlockSpec(shape, lambda i: (0, 0))


_mlp_call = pl.pallas_call(
    _mlp_body,
    grid=(B // BLK,),
    in_specs=[
        pl.BlockSpec((BLK, D), lambda i: (i, 0)),
        pl.BlockSpec((BLK, D), lambda i: (i, 0)),
        pl.BlockSpec((BLK, 16), lambda i: (i, 0)),
        pl.BlockSpec((BLK, 1), lambda i: (i, 0)),
        pl.BlockSpec((BLK, 1), lambda i: (i, 0)),
        pl.BlockSpec((BLK, 1), lambda i: (i, 0)),
        _full((2, 8)), _full((7, 8)), _full((21, 8)),
        _full((104, 128)), _full((1, 128)),
        _full((128, 64)), _full((1, 64)),
        _full((64, 32)), _full((1, 32)),
        _full((32, 1)), _full((1, 1)),
    ],
    out_specs=pl.BlockSpec((BLK, 1), lambda i: (i, 0)),
    out_shape=jax.ShapeDtypeStruct((B, 1), jnp.float32),
)


def kernel(user_id, movie_id, gender, age, occupation, continuous_features,
           user_table, movie_table, gender_table, age_table, occupation_table,
           W0, b0, W1, b1, W2, b2, W3, b3):
    info = plsc.get_sparse_core_info()
    NW = info.num_cores * info.num_subcores
    n_chunks = (B // NW) // IDX_CHUNK
    uid = user_id.astype(jnp.int32).reshape(NW, n_chunks, IDX_CHUNK)
    mid = movie_id.astype(jnp.int32).reshape(NW, n_chunks, IDX_CHUNK)
    U, M = _make_gather(info.num_cores, info.num_subcores)(
        user_table, movie_table, uid, mid)
    return _mlp_call(
        U, M, continuous_features,
        gender.astype(jnp.int32).reshape(B, 1),
        age.astype(jnp.int32).reshape(B, 1),
        occupation.astype(jnp.int32).reshape(B, 1),
        gender_table, age_table, occupation_table,
        W0, b0.reshape(1, 128), W1, b1.reshape(1, 64),
        W2, b2.reshape(1, 32), W3, b3.reshape(1, 1))
